# Initial kernel scaffold; baseline (speedup 1.0000x reference)
import jax
import jax.numpy as jnp
from jax.experimental import pallas as pl


def kernel(z, emb, W, b):
    raise NotImplementedError("write your pallas kernel here")


# speedup vs baseline: 1.3749x; 1.3749x over previous
import functools

import jax
import jax.numpy as jnp
from jax import lax
from jax.experimental import pallas as pl
from jax.experimental.pallas import tpu as pltpu
from jax.experimental.pallas import tpu_sc as plsc

_N = 16384
_D = 64
_K = 8192
_DH = 32
_EPS = 1e-12

_BN = 256

_NC = 2
_NS = 16
_NW = _NC * _NS


def _prep_body(emb_ref, wt_ref, b_ref, embA_ref, embN_ref):
    emb = emb_ref[...]
    e = jnp.dot(emb, wt_ref[...],
                preferred_element_type=jnp.float32) + b_ref[...]
    n2 = jnp.sum(e * e, axis=1, keepdims=True)
    en = e / jnp.maximum(jnp.sqrt(n2), _EPS)
    e2 = jnp.sum(en * en, axis=1, keepdims=True)
    embA_ref[...] = jnp.concatenate([en, -0.5 * e2], axis=1)
    m2 = jnp.sum(emb * emb, axis=1, keepdims=True)
    embN_ref[...] = emb / jnp.maximum(jnp.sqrt(m2), _EPS)


def _argmin_body(z_ref, wt_ref, b_ref, embAT_ref, idx_ref):
    zb = z_ref[...]
    e = jnp.dot(zb, wt_ref[...],
                preferred_element_type=jnp.float32) + b_ref[...]
    n2 = jnp.sum(e * e, axis=1, keepdims=True)
    zn = e / jnp.maximum(jnp.sqrt(n2), _EPS)
    za = jnp.concatenate(
        [zn, jnp.ones((zn.shape[0], 1), jnp.float32)], axis=1)
    s = jnp.dot(za, embAT_ref[...],
                preferred_element_type=jnp.float32)
    m = jnp.max(s, axis=1, keepdims=True)
    ji = lax.broadcasted_iota(jnp.int32, s.shape, 1)
    cand = jnp.where(s == m, ji, _K)
    idx_ref[0, 0, :] = jnp.min(cand, axis=1)


def _prep_call(emb, wt, b2):
    return pl.pallas_call(
        _prep_body,
        out_shape=(
            jax.ShapeDtypeStruct((_K, _DH + 1), jnp.float32),
            jax.ShapeDtypeStruct((_K, _D), jnp.float32),
        ),
    )(emb, wt, b2)


def _argmin_call(z, wt, b2, embAT):
    nb = _N // _BN
    return pl.pallas_call(
        _argmin_body,
        grid=(nb,),
        in_specs=[
            pl.BlockSpec((_BN, _D), lambda i: (i, 0)),
            pl.BlockSpec((_D, _DH), lambda i: (0, 0)),
            pl.BlockSpec((1, _DH), lambda i: (0, 0)),
            pl.BlockSpec((_DH + 1, _K), lambda i: (0, 0)),
        ],
        out_specs=pl.BlockSpec((1, 1, _BN), lambda i: (i, 0, 0)),
        out_shape=jax.ShapeDtypeStruct((nb, 1, _BN), jnp.int32),
    )(z, wt, b2, embAT)


def _gather_call(table, idx):
    b_per_w = _N // _NW
    mesh = plsc.VectorSubcoreMesh(core_axis_name="c", subcore_axis_name="s")

    @functools.partial(
        pl.kernel,
        mesh=mesh,
        compiler_params=pltpu.CompilerParams(use_tc_tiling_on_sc=False),
        out_type=jax.ShapeDtypeStruct((_N, _D), jnp.float32),
        scratch_types=[
            pltpu.VMEM((b_per_w,), jnp.int32),
            pltpu.VMEM((b_per_w, _D), jnp.float32),
            pltpu.SemaphoreType.DMA,
        ],
    )
    def k(table_hbm, idx_hbm, out_hbm, idx_v, rows_v, sem):
        wid = lax.axis_index("s") * _NC + lax.axis_index("c")
        base = wid * b_per_w
        pltpu.sync_copy(idx_hbm.at[pl.ds(base, b_per_w)], idx_v)
        pltpu.async_copy(table_hbm.at[idx_v], rows_v, sem).wait()
        pltpu.sync_copy(rows_v, out_hbm.at[pl.ds(base, b_per_w)])

    return k(table, idx)


def kernel(z, emb, W, b):
    assert z.shape == (_N, _D) and emb.shape == (_K, _D)
    wt = W.T
    b2 = b.reshape(1, _DH)
    embA, embN = _prep_call(emb, wt, b2)
    idx3 = _argmin_call(z, wt, b2, embA.T)
    idx = idx3.reshape(_N)
    return _gather_call(embN, idx)
